# baseline (speedup 1.0000x reference)
import functools

import jax
import jax.numpy as jnp
from jax import lax
from jax.experimental import pallas as pl
from jax.experimental.pallas import tpu as pltpu
from jax.experimental.pallas import tpu_sc as plsc

B, C, D, H, W = 2, 8, 128, 128, 128
EPS = 1e-05
DHW = D * H * W
ROWS = B * C
NC, NS, L = 2, 16, 16
NW = NC * NS
K = 2048
STEPS = K // L

NCH = 10
SPAN = K * NCH
SC_COLS = NW * SPAN
TC_COLS = DHW - SC_COLS
BWT = 1024
NBALL = DHW // BWT
NB_TC = TC_COLS // BWT


def _dice_body(out_hbm, tgt_hbm, part_hbm,
               ob0, ob1, tb0, tb1, bins, cnt,
               prow, tmp, tot, stage, res, shared,
               so0, so1, st0, st1):
    cid = lax.axis_index("c")
    sid = lax.axis_index("s")
    wid = sid * NC + cid
    base = TC_COLS + wid * SPAN

    obufs = (ob0, ob1)
    tbufs = (tb0, tb1)
    osems = (so0, so1)
    tsems = (st0, st1)

    zero = jnp.zeros((L,), jnp.float32)
    for r in range(ROWS):
        bins[r, :] = zero
        cnt[r, :] = zero

    def start(s, off):
        pltpu.async_copy(out_hbm.at[:, pl.ds(off, K)], obufs[s], osems[s])
        pltpu.async_copy(tgt_hbm.at[:, pl.ds(off, K)], tbufs[s], tsems[s])

    def wait(s):
        pltpu.make_async_copy(
            out_hbm.at[:, pl.ds(0, K)], obufs[s], osems[s]).wait()
        pltpu.make_async_copy(
            tgt_hbm.at[:, pl.ds(0, K)], tbufs[s], tsems[s]).wait()

    start(0, base)
    start(1, base + K)

    iota = lax.iota(jnp.int32, L)
    ones = jnp.full((L,), 1.0, jnp.float32)

    def run_chunk(s, accs):
        ob, tb = obufs[s], tbufs[s]

        def step(i, accs):
            lo = i * L
            col = iota + lo
            accs = list(accs)
            for b in range(B):
                t = tb[b, pl.ds(lo, L)]
                row = t + (C * b) if b else t
                g = plsc.load_gather(ob, [row, col])
                plsc.addupdate_scatter(bins, [row, iota], g)
                plsc.addupdate_scatter(cnt, [row, iota], ones)
                for c in range(C):
                    r = C * b + c
                    accs[r] = accs[r] + ob[r, pl.ds(lo, L)]
            return tuple(accs)

        return plsc.parallel_loop(
            0, STEPS, 1, unroll=4, carry=tuple(accs))(step)

    def chunk_iter(j, accs):
        for s in range(2):
            wait(s)
            accs = run_chunk(s, accs)

            @pl.when(j < NCH // 2 - 1)
            def _():
                start(s, base + (j * 2 + s + 2) * K)
        return accs

    accs0 = tuple(jnp.zeros((L,), jnp.float32) for _ in range(ROWS))
    accs = lax.fori_loop(0, NCH // 2, chunk_iter, accs0)

    for r in range(ROWS):
        prow[r, :] = accs[r]
        prow[ROWS + r, :] = bins[r, :]
        prow[2 * ROWS + r, :] = cnt[r, :]
    pltpu.sync_copy(prow, shared.at[sid])
    plsc.subcore_barrier()

    @pl.when(sid == 0)
    def _():
        pltpu.sync_copy(shared.at[0], tot)
        for w in range(1, NS):
            pltpu.sync_copy(shared.at[w], tmp)
            for r in range(3 * ROWS):
                tot[r, :] = tot[r, :] + tmp[r, :]
        for blk in range(3):
            rs = jnp.zeros((L,), jnp.float32)
            rows = iota + blk * ROWS
            for j in range(L):
                rs = rs + plsc.load_gather(
                    tot, [rows, jnp.full((L,), j, jnp.int32)])
            stage[blk, :] = rs
        sw = iota ^ C
        for blk in range(3):
            res[blk, :] = stage[blk, :] + plsc.load_gather(
                stage, [jnp.full((L,), blk, jnp.int32), sw])
        pltpu.sync_copy(res, part_hbm.at[cid])


@functools.partial(
    pl.kernel,
    out_type=jax.ShapeDtypeStruct((NC, 3, L), jnp.float32),
    mesh=plsc.VectorSubcoreMesh(
        core_axis_name="c", subcore_axis_name="s",
        num_cores=NC, num_subcores=NS),
    scratch_types=[
        pltpu.VMEM((ROWS, K), jnp.float32),
        pltpu.VMEM((ROWS, K), jnp.float32),
        pltpu.VMEM((B, K), jnp.int32),
        pltpu.VMEM((B, K), jnp.int32),
        pltpu.VMEM((ROWS, L), jnp.float32),
        pltpu.VMEM((ROWS, L), jnp.float32),
        pltpu.VMEM((3 * ROWS, L), jnp.float32),
        pltpu.VMEM((3 * ROWS, L), jnp.float32),
        pltpu.VMEM((3 * ROWS, L), jnp.float32),
        pltpu.VMEM((3, L), jnp.float32),
        pltpu.VMEM((3, L), jnp.float32),
        pltpu.VMEM_SHARED((NS, 3 * ROWS, L), jnp.float32),
        pltpu.SemaphoreType.DMA,
        pltpu.SemaphoreType.DMA,
        pltpu.SemaphoreType.DMA,
        pltpu.SemaphoreType.DMA,
    ],
    compiler_params=pltpu.CompilerParams(
        use_tc_tiling_on_sc=False, needs_layout_passes=False),
)
def _dice_partials(out_hbm, tgt_hbm, part_hbm, *scratch):
    _dice_body(out_hbm, tgt_hbm, part_hbm, *scratch)


def _tc_body(o_ref, t0_ref, t1_ref, out_ref, acc):
    g = pl.program_id(0)

    @pl.when(g == 0)
    def _():
        acc[...] = jnp.zeros_like(acc)

    tv = (t0_ref[0], t1_ref[0])
    for b in range(B):
        for c in range(C):
            r = C * b + c
            x = o_ref[r, 0]
            m = tv[b] == c
            acc[0, r] = acc[0, r] + x
            acc[1, r] = acc[1, r] + jnp.where(m, x, 0.0)
            acc[2, r] = acc[2, r] + jnp.where(m, 1.0, 0.0)

    @pl.when(g == NB_TC - 1)
    def _():
        out_ref[...] = jnp.sum(acc[...], axis=2)


_dice_tc = pl.pallas_call(
    _tc_body,
    grid=(NB_TC,),
    in_specs=[
        pl.BlockSpec((ROWS, 1, 8, 128), lambda g: (0, g, 0, 0)),
        pl.BlockSpec((1, 8, 128), lambda g: (g, 0, 0)),
        pl.BlockSpec((1, 8, 128), lambda g: (NBALL + g, 0, 0)),
    ],
    out_specs=pl.BlockSpec((3, ROWS, 128), lambda g: (0, 0, 0)),
    out_shape=jax.ShapeDtypeStruct((3, ROWS, 128), jnp.float32),
    scratch_shapes=[pltpu.VMEM((3, ROWS, 8, 128), jnp.float32)],
    compiler_params=pltpu.CompilerParams(
        dimension_semantics=("arbitrary",)),
)


@jax.jit
def kernel(output, target):
    out2d = output.reshape(ROWS, DHW)
    tgt2d = target.reshape(B, DHW)
    part = _dice_partials(out2d, tgt2d)
    out4d = output.reshape(ROWS, NBALL, 8, 128)
    tgtr = tgt2d.reshape(B * NBALL, 8, 128)
    ptc = _dice_tc(out4d, tgtr, tgtr)
    tsc = (part[0] + part[1])[:, :C]
    ttc3 = ptc.sum(-1)
    ttc = ttc3[:, :C] + ttc3[:, C:]
    tot = tsc + ttc
    dice = 2.0 * tot[1] / jnp.maximum(tot[0] + tot[2], EPS)
    return (dice, jnp.mean(dice))


# speedup vs baseline: 1.0015x; 1.0015x over previous
import functools

import jax
import jax.numpy as jnp
from jax import lax
from jax.experimental import pallas as pl
from jax.experimental.pallas import tpu as pltpu
from jax.experimental.pallas import tpu_sc as plsc

B, C, D, H, W = 2, 8, 128, 128, 128
EPS = 1e-05
DHW = D * H * W
ROWS = B * C
NC, NS, L = 2, 16, 16
NW = NC * NS
K = 2048
STEPS = K // L

NCH = 10
SPAN = K * NCH
SC_COLS = NW * SPAN
TC_COLS = DHW - SC_COLS
BWT = 1024
NBALL = DHW // BWT
NB_TC = TC_COLS // BWT


def _dice_body(out_hbm, tgt_hbm, part_hbm,
               ob0, ob1, tb0, tb1, bins, cnt,
               prow, tmp, tot, stage, res, shared,
               so0, so1, st0, st1):
    cid = lax.axis_index("c")
    sid = lax.axis_index("s")
    wid = sid * NC + cid
    base = TC_COLS + wid * SPAN

    obufs = (ob0, ob1)
    tbufs = (tb0, tb1)
    osems = (so0, so1)
    tsems = (st0, st1)

    zero = jnp.zeros((L,), jnp.float32)
    for r in range(ROWS):
        bins[r, :] = zero
        cnt[r, :] = zero

    def start(s, off):
        pltpu.async_copy(out_hbm.at[:, pl.ds(off, K)], obufs[s], osems[s])
        pltpu.async_copy(tgt_hbm.at[:, pl.ds(off, K)], tbufs[s], tsems[s])

    def wait(s):
        pltpu.make_async_copy(
            out_hbm.at[:, pl.ds(0, K)], obufs[s], osems[s]).wait()
        pltpu.make_async_copy(
            tgt_hbm.at[:, pl.ds(0, K)], tbufs[s], tsems[s]).wait()

    start(0, base)
    start(1, base + K)

    iota = lax.iota(jnp.int32, L)
    ones = jnp.full((L,), 1.0, jnp.float32)

    def run_chunk(s, accs):
        ob, tb = obufs[s], tbufs[s]

        def step(i, accs):
            lo = i * L
            col = iota + lo
            accs = list(accs)
            for b in range(B):
                t = tb[b, pl.ds(lo, L)]
                row = t + (C * b) if b else t
                g = plsc.load_gather(ob, [row, col])
                plsc.addupdate_scatter(bins, [row, iota], g)
                plsc.addupdate_scatter(cnt, [row, iota], ones)
                for c in range(C):
                    r = C * b + c
                    accs[r] = accs[r] + ob[r, pl.ds(lo, L)]
            return tuple(accs)

        return plsc.parallel_loop(
            0, STEPS, 1, unroll=4, carry=tuple(accs))(step)

    def chunk_iter(j, accs):
        for s in range(2):
            wait(s)
            accs = run_chunk(s, accs)

            @pl.when(j < NCH // 2 - 1)
            def _():
                start(s, base + (j * 2 + s + 2) * K)
        return accs

    accs0 = tuple(jnp.zeros((L,), jnp.float32) for _ in range(ROWS))
    accs = lax.fori_loop(0, NCH // 2, chunk_iter, accs0)

    for r in range(ROWS):
        prow[r, :] = accs[r]
        prow[ROWS + r, :] = bins[r, :]
        prow[2 * ROWS + r, :] = cnt[r, :]
    pltpu.sync_copy(prow, shared.at[sid])
    plsc.subcore_barrier()

    @pl.when(sid == 0)
    def _():
        pltpu.sync_copy(shared.at[0], tot)
        for w in range(1, NS):
            pltpu.sync_copy(shared.at[w], tmp)
            for r in range(3 * ROWS):
                tot[r, :] = tot[r, :] + tmp[r, :]
        for blk in range(3):
            rs = jnp.zeros((L,), jnp.float32)
            rows = iota + blk * ROWS
            for j in range(L):
                rs = rs + plsc.load_gather(
                    tot, [rows, jnp.full((L,), j, jnp.int32)])
            stage[blk, :] = rs
        sw = iota ^ C
        for blk in range(3):
            res[blk, :] = stage[blk, :] + plsc.load_gather(
                stage, [jnp.full((L,), blk, jnp.int32), sw])
        pltpu.sync_copy(res, part_hbm.at[cid])


@functools.partial(
    pl.kernel,
    out_type=jax.ShapeDtypeStruct((NC, 3, L), jnp.float32),
    mesh=plsc.VectorSubcoreMesh(
        core_axis_name="c", subcore_axis_name="s",
        num_cores=NC, num_subcores=NS),
    scratch_types=[
        pltpu.VMEM((ROWS, K), jnp.float32),
        pltpu.VMEM((ROWS, K), jnp.float32),
        pltpu.VMEM((B, K), jnp.int32),
        pltpu.VMEM((B, K), jnp.int32),
        pltpu.VMEM((ROWS, L), jnp.float32),
        pltpu.VMEM((ROWS, L), jnp.float32),
        pltpu.VMEM((3 * ROWS, L), jnp.float32),
        pltpu.VMEM((3 * ROWS, L), jnp.float32),
        pltpu.VMEM((3 * ROWS, L), jnp.float32),
        pltpu.VMEM((3, L), jnp.float32),
        pltpu.VMEM((3, L), jnp.float32),
        pltpu.VMEM_SHARED((NS, 3 * ROWS, L), jnp.float32),
        pltpu.SemaphoreType.DMA,
        pltpu.SemaphoreType.DMA,
        pltpu.SemaphoreType.DMA,
        pltpu.SemaphoreType.DMA,
    ],
    compiler_params=pltpu.CompilerParams(
        use_tc_tiling_on_sc=False, needs_layout_passes=False),
)
def _dice_partials(out_hbm, tgt_hbm, part_hbm, *scratch):
    _dice_body(out_hbm, tgt_hbm, part_hbm, *scratch)


B2 = 16
NJ_TC = TC_COLS // (B2 * BWT)


def _tc_body(o_ref, t_ref, out_ref, acc):
    j = pl.program_id(0)
    r = pl.program_id(1)

    @pl.when((j == 0) & (r == 0))
    def _():
        acc[...] = jnp.zeros_like(acc)

    cls = r % C
    x = o_ref[0]
    t = t_ref[...]
    m = t == cls
    osum_p = jnp.sum(x, axis=0)
    inter_p = jnp.sum(jnp.where(m, x, 0.0), axis=0)
    cnt_p = jnp.sum(jnp.where(m, 1.0, 0.0), axis=0)
    acc[0, r] = acc[0, r] + osum_p
    acc[1, r] = acc[1, r] + inter_p
    acc[2, r] = acc[2, r] + cnt_p

    @pl.when((j == NJ_TC - 1) & (r == ROWS - 1))
    def _():
        out_ref[...] = jnp.sum(acc[...], axis=2)


_dice_tc = pl.pallas_call(
    _tc_body,
    grid=(NJ_TC, ROWS),
    in_specs=[
        pl.BlockSpec((1, B2, 8, 128), lambda j, r: (r, j, 0, 0)),
        pl.BlockSpec((B2, 8, 128),
                     lambda j, r: ((r // C) * (NBALL // B2) + j, 0, 0)),
    ],
    out_specs=pl.BlockSpec((3, ROWS, 128), lambda j, r: (0, 0, 0)),
    out_shape=jax.ShapeDtypeStruct((3, ROWS, 128), jnp.float32),
    scratch_shapes=[pltpu.VMEM((3, ROWS, 8, 128), jnp.float32)],
    compiler_params=pltpu.CompilerParams(
        dimension_semantics=("arbitrary", "arbitrary")),
)


@jax.jit
def kernel(output, target):
    out2d = output.reshape(ROWS, DHW)
    tgt2d = target.reshape(B, DHW)
    part = _dice_partials(out2d, tgt2d)
    out4d = output.reshape(ROWS, NBALL, 8, 128)
    tgtr = tgt2d.reshape(B * NBALL, 8, 128)
    ptc = _dice_tc(out4d, tgtr)
    tsc = (part[0] + part[1])[:, :C]
    ttc3 = ptc.sum(-1)
    ttc = ttc3[:, :C] + ttc3[:, C:]
    tot = tsc + ttc
    dice = 2.0 * tot[1] / jnp.maximum(tot[0] + tot[2], EPS)
    return (dice, jnp.mean(dice))
